# baseline (speedup 1.0000x reference)
import functools

import jax
import jax.numpy as jnp
import numpy as np
from jax import lax
from jax.experimental import pallas as pl
from jax.experimental.pallas import tpu as pltpu
from jax.experimental.pallas import tpu_sc as plsc

RADIUS = 1
NNB = 27
FEAT = 128
GRID = 64
CENTER = 13

NC = 2
NS = 16
NW = NC * NS
CH = 128


def _shift_grid():
    r = np.arange(-RADIUS, RADIUS + 1)
    return np.stack(np.meshgrid(r, r, r, indexing="ij"), axis=-1).reshape(-1, 3)


@functools.partial(jax.jit, static_argnums=(2,))
def _sc_gather(table, ids, n_ch):
    rows_per_w = n_ch * CH
    total = NW * rows_per_w

    def body(table_hbm, ids_hbm, out_hbm, idx_v, buf, gsem):
        wid = lax.axis_index("s") * NC + lax.axis_index("c")
        base = wid * rows_per_w
        pltpu.sync_copy(ids_hbm.at[wid], idx_v)

        def step(j, carry):
            pltpu.async_copy(table_hbm.at[idx_v.at[j]], buf, gsem).wait()
            pltpu.sync_copy(buf, out_hbm.at[pl.ds(base + j * CH, CH)])
            return carry

        lax.fori_loop(0, n_ch, step, 0, unroll=False)

    mesh = plsc.VectorSubcoreMesh(core_axis_name="c", subcore_axis_name="s")
    f = pl.kernel(
        body,
        out_type=jax.ShapeDtypeStruct((total, FEAT), jnp.float32),
        mesh=mesh,
        scratch_types=[
            pltpu.VMEM((n_ch, CH), jnp.int32),
            pltpu.VMEM((CH, FEAT), jnp.float32),
            pltpu.SemaphoreType.DMA,
        ],
    )
    return f(table, ids)


def _tc_matmul(gathered, mask, wt, b, block_n):
    n = gathered.shape[0]

    def body(g_ref, m_ref, wt_ref, b_ref, out_ref):
        g = g_ref[...]
        m = m_ref[...]
        parts = [
            g[:, k * FEAT:(k + 1) * FEAT] * m[:, k:k + 1]
            for k in range(NNB)
        ]
        gm = jnp.concatenate(parts, axis=1)
        acc = jax.lax.dot_general(
            gm, wt_ref[...], (((1,), (0,)), ((), ())),
            preferred_element_type=jnp.float32)
        out_ref[...] = jnp.concatenate(
            [g[:, CENTER * FEAT:(CENTER + 1) * FEAT], acc + b_ref[...]],
            axis=1)

    return pl.pallas_call(
        body,
        grid=(n // block_n,),
        in_specs=[
            pl.BlockSpec((block_n, NNB * FEAT), lambda i: (i, 0)),
            pl.BlockSpec((block_n, NNB), lambda i: (i, 0)),
            pl.BlockSpec((NNB * FEAT, FEAT), lambda i: (0, 0)),
            pl.BlockSpec((1, FEAT), lambda i: (0, 0)),
        ],
        out_specs=pl.BlockSpec((block_n, 2 * FEAT), lambda i: (i, 0)),
        out_shape=jax.ShapeDtypeStruct((n, 2 * FEAT), jnp.float32),
    )(gathered, mask, wt, b)


def kernel(query_indices, query_points, feature_volume, count_volume, W, b):
    del query_points, count_volume
    qi = query_indices.reshape(-1, 3)
    n = qi.shape[0]

    shift = jnp.asarray(_shift_grid(), dtype=jnp.int32)
    nb = qi[:, None, :] + shift[None, :, :]
    valid = jnp.all((nb >= 0) & (nb < GRID), axis=-1)
    nbc = jnp.clip(nb, 0, GRID - 1)
    ids = (nbc[..., 0] * GRID + nbc[..., 1]) * GRID + nbc[..., 2]
    mask = valid.astype(jnp.float32)

    rows = n * NNB
    rows_per_w = rows // NW
    n_ch = rows_per_w // CH
    ids_flat = ids.reshape(NW, n_ch, CH)

    table = feature_volume.reshape(GRID * GRID * GRID, FEAT)
    gathered = _sc_gather(table, ids_flat, n_ch)
    gathered = gathered.reshape(n, NNB * FEAT)

    wt = W.T
    out = _tc_matmul(gathered, mask, wt, b.reshape(1, FEAT), block_n=512)
    return (out, qi)


# speedup vs baseline: 1.1287x; 1.1287x over previous
import functools

import jax
import jax.numpy as jnp
import numpy as np
from jax import lax
from jax.experimental import pallas as pl
from jax.experimental.pallas import tpu as pltpu
from jax.experimental.pallas import tpu_sc as plsc

RADIUS = 1
NNB = 27
FEAT = 128
GRID = 64
CENTER = 13

NC = 2
NS = 16
NW = NC * NS
CH = 128


def _shift_grid():
    r = np.arange(-RADIUS, RADIUS + 1)
    return np.stack(np.meshgrid(r, r, r, indexing="ij"), axis=-1).reshape(-1, 3)


NBUF = 4


@functools.partial(jax.jit, static_argnums=(2,))
def _sc_gather(table, ids, n_ch):
    rows_per_w = n_ch * CH
    total = NW * rows_per_w
    n_outer = n_ch // NBUF

    def body(table_hbm, ids_hbm, out_hbm, idx_v, bufs, gsem, wsem):
        wid = lax.axis_index("s") * NC + lax.axis_index("c")
        base = wid * rows_per_w
        pltpu.sync_copy(ids_hbm.at[wid], idx_v)

        def start_g(j, b):
            pltpu.async_copy(table_hbm.at[idx_v.at[j]], bufs.at[b], gsem.at[b])

        def wait_g(b):
            pltpu.make_async_copy(
                table_hbm.at[idx_v.at[0]], bufs.at[b], gsem.at[b]).wait()

        def start_w(j, b):
            pltpu.async_copy(
                bufs.at[b], out_hbm.at[pl.ds(base + j * CH, CH)], wsem.at[b])

        def wait_w(b):
            pltpu.make_async_copy(
                bufs.at[b], out_hbm.at[pl.ds(base, CH)], wsem.at[b]).wait()

        for b in range(NBUF):
            start_g(b, b)

        def outer(t, carry):
            j0 = t * NBUF
            for b in range(NBUF):
                wait_g(b)
                start_w(j0 + b, b)
            for b in range(NBUF):
                wait_w(b)
                start_g(j0 + NBUF + b, b)
            return carry

        lax.fori_loop(0, n_outer - 1, outer, 0, unroll=False)

        j0 = (n_outer - 1) * NBUF
        for b in range(NBUF):
            wait_g(b)
            start_w(j0 + b, b)
        for b in range(NBUF):
            wait_w(b)

    mesh = plsc.VectorSubcoreMesh(core_axis_name="c", subcore_axis_name="s")
    f = pl.kernel(
        body,
        out_type=jax.ShapeDtypeStruct((total, FEAT), jnp.float32),
        mesh=mesh,
        scratch_types=[
            pltpu.VMEM((n_ch, CH), jnp.int32),
            pltpu.VMEM((NBUF, CH, FEAT), jnp.float32),
            pltpu.SemaphoreType.DMA((NBUF,)),
            pltpu.SemaphoreType.DMA((NBUF,)),
        ],
    )
    return f(table, ids)


def _tc_matmul(gathered, mask, wt, b, block_n):
    n = gathered.shape[0]

    def body(g_ref, m_ref, wt_ref, b_ref, out_ref):
        g = g_ref[...]
        m = m_ref[...]
        bn = g.shape[0]
        mx = jnp.broadcast_to(m[:, :, None], (bn, NNB, FEAT))
        mx = mx.reshape(bn, NNB * FEAT)
        gm = (g * mx).astype(jnp.bfloat16)
        acc = jax.lax.dot_general(
            gm, wt_ref[...], (((1,), (0,)), ((), ())),
            preferred_element_type=jnp.float32)
        out_ref[...] = jnp.concatenate(
            [g[:, CENTER * FEAT:(CENTER + 1) * FEAT], acc + b_ref[...]],
            axis=1)

    return pl.pallas_call(
        body,
        grid=(n // block_n,),
        in_specs=[
            pl.BlockSpec((block_n, NNB * FEAT), lambda i: (i, 0)),
            pl.BlockSpec((block_n, NNB), lambda i: (i, 0)),
            pl.BlockSpec((NNB * FEAT, FEAT), lambda i: (0, 0)),
            pl.BlockSpec((1, FEAT), lambda i: (0, 0)),
        ],
        out_specs=pl.BlockSpec((block_n, 2 * FEAT), lambda i: (i, 0)),
        out_shape=jax.ShapeDtypeStruct((n, 2 * FEAT), jnp.float32),
    )(gathered, mask, wt, b)


def kernel(query_indices, query_points, feature_volume, count_volume, W, b):
    del query_points, count_volume
    qi = query_indices.reshape(-1, 3)
    n = qi.shape[0]

    shift = jnp.asarray(_shift_grid(), dtype=jnp.int32)
    nb = qi[:, None, :] + shift[None, :, :]
    valid = jnp.all((nb >= 0) & (nb < GRID), axis=-1)
    nbc = jnp.clip(nb, 0, GRID - 1)
    ids = (nbc[..., 0] * GRID + nbc[..., 1]) * GRID + nbc[..., 2]
    mask = valid.astype(jnp.float32)

    rows = n * NNB
    rows_per_w = rows // NW
    n_ch = rows_per_w // CH
    ids_flat = ids.reshape(NW, n_ch, CH)

    table = feature_volume.reshape(GRID * GRID * GRID, FEAT)
    gathered = _sc_gather(table, ids_flat, n_ch)
    gathered = gathered.reshape(n, NNB * FEAT)

    wt = W.T.astype(jnp.bfloat16)
    out = _tc_matmul(gathered, mask, wt, b.reshape(1, FEAT), block_n=512)
    return (out, qi)


# speedup vs baseline: 3.2292x; 2.8610x over previous
import functools

import jax
import jax.numpy as jnp
import numpy as np
from jax import lax
from jax.experimental import pallas as pl
from jax.experimental.pallas import tpu as pltpu
from jax.experimental.pallas import tpu_sc as plsc

RADIUS = 1
NNB = 27
FEAT = 128
GRID = 64
CENTER = 13

NC = 2
NS = 16
NW = NC * NS
CH = 128


def _shift_grid():
    r = np.arange(-RADIUS, RADIUS + 1)
    return np.stack(np.meshgrid(r, r, r, indexing="ij"), axis=-1).reshape(-1, 3)


NBUF = 4


@functools.partial(jax.jit, static_argnums=(2,))
def _sc_gather(table, ids, n_ch):
    rows_per_w = n_ch * CH
    total = NW * rows_per_w
    n_outer = n_ch // NBUF

    def body(table_hbm, ids_hbm, out_hbm, idx_v, bufs, gsem, wsem):
        wid = lax.axis_index("s") * NC + lax.axis_index("c")
        base = wid * rows_per_w
        pltpu.sync_copy(ids_hbm.at[wid], idx_v)

        def start_g(j, b):
            pltpu.async_copy(table_hbm.at[idx_v.at[j]], bufs.at[b], gsem.at[b])

        def wait_g(b):
            pltpu.make_async_copy(
                table_hbm.at[idx_v.at[0]], bufs.at[b], gsem.at[b]).wait()

        def start_w(j, b):
            pltpu.async_copy(
                bufs.at[b], out_hbm.at[pl.ds(base + j * CH, CH)], wsem.at[b])

        def wait_w(b):
            pltpu.make_async_copy(
                bufs.at[b], out_hbm.at[pl.ds(base, CH)], wsem.at[b]).wait()

        for b in range(NBUF):
            start_g(b, b)

        def outer(t, carry):
            j0 = t * NBUF
            for b in range(NBUF):
                wait_g(b)
                start_w(j0 + b, b)
            for b in range(NBUF):
                wait_w(b)
                start_g(j0 + NBUF + b, b)
            return carry

        lax.fori_loop(0, n_outer - 1, outer, 0, unroll=False)

        j0 = (n_outer - 1) * NBUF
        for b in range(NBUF):
            wait_g(b)
            start_w(j0 + b, b)
        for b in range(NBUF):
            wait_w(b)

    mesh = plsc.VectorSubcoreMesh(core_axis_name="c", subcore_axis_name="s")
    f = pl.kernel(
        body,
        out_type=jax.ShapeDtypeStruct((total, FEAT), jnp.float32),
        mesh=mesh,
        scratch_types=[
            pltpu.VMEM((n_ch, CH), jnp.int32),
            pltpu.VMEM((NBUF, CH, FEAT), jnp.float32),
            pltpu.SemaphoreType.DMA((NBUF,)),
            pltpu.SemaphoreType.DMA((NBUF,)),
        ],
    )
    return f(table, ids)


def _tc_matmul(gathered, mask, wt, b, block_n):
    n = gathered.shape[0]

    def body(g_ref, m_ref, wt_ref, b_ref, out_ref):
        m = m_ref[...]
        acc = jnp.broadcast_to(b_ref[...], (block_n, FEAT))
        for k in range(NNB):
            gk = (g_ref[k] * m[:, k:k + 1]).astype(jnp.bfloat16)
            acc = acc + jax.lax.dot_general(
                gk, wt_ref[k], (((1,), (0,)), ((), ())),
                preferred_element_type=jnp.float32)
        out_ref[...] = jnp.concatenate([g_ref[CENTER], acc], axis=1)

    return pl.pallas_call(
        body,
        grid=(n // block_n,),
        in_specs=[
            pl.BlockSpec((NNB, block_n, FEAT), lambda i: (0, i, 0)),
            pl.BlockSpec((block_n, NNB), lambda i: (i, 0)),
            pl.BlockSpec((NNB, FEAT, FEAT), lambda i: (0, 0, 0)),
            pl.BlockSpec((1, FEAT), lambda i: (0, 0)),
        ],
        out_specs=pl.BlockSpec((block_n, 2 * FEAT), lambda i: (i, 0)),
        out_shape=jax.ShapeDtypeStruct((n, 2 * FEAT), jnp.float32),
    )(gathered, mask, wt, b)


def kernel(query_indices, query_points, feature_volume, count_volume, W, b):
    del query_points, count_volume
    qi = query_indices.reshape(-1, 3)
    n = qi.shape[0]

    shift = jnp.asarray(_shift_grid(), dtype=jnp.int32)
    nb = qi[:, None, :] + shift[None, :, :]
    valid = jnp.all((nb >= 0) & (nb < GRID), axis=-1)
    nbc = jnp.clip(nb, 0, GRID - 1)
    ids = (nbc[..., 0] * GRID + nbc[..., 1]) * GRID + nbc[..., 2]
    mask = valid.astype(jnp.float32)

    rows = n * NNB
    rows_per_w = rows // NW
    n_ch = rows_per_w // CH
    ids_flat = ids.T.reshape(NW, n_ch, CH)

    table = feature_volume.reshape(GRID * GRID * GRID, FEAT)
    gathered = _sc_gather(table, ids_flat, n_ch)
    gathered = gathered.reshape(NNB, n, FEAT)

    wt3 = W.T.reshape(NNB, FEAT, FEAT).astype(jnp.bfloat16)
    out = _tc_matmul(gathered, mask, wt3, b.reshape(1, FEAT), block_n=512)
    return (out, qi)
